# Initial kernel scaffold; baseline (speedup 1.0000x reference)
import jax
import jax.numpy as jnp
from jax.experimental import pallas as pl


def kernel(h, edge_index, e, W, b, eps, gamma, beta, Wp, bp):
    raise NotImplementedError("write your pallas kernel here")


# speedup vs baseline: 4.2596x; 4.2596x over previous
import functools

import jax
import jax.numpy as jnp
from jax import lax
from jax.experimental import pallas as pl
from jax.experimental.pallas import tpu as pltpu
from jax.experimental.pallas import tpu_sc as plsc

N = 10000
E = 320000
D = 128
L = 4
C = 10

try:
    _info = plsc.get_sparse_core_info()
    NC = _info.num_cores
    NS = _info.num_subcores
except Exception:
    NC, NS = 2, 16
NW = NC * NS
EPW = E // NW
CH = 80
NCHUNK = EPW // CH
NP = 10240
RPS = NP // NS

@functools.cache
def _make_sc_agg():
    mesh = plsc.VectorSubcoreMesh(core_axis_name="c", subcore_axis_name="s")

    @functools.partial(
        pl.kernel,
        out_type=jax.ShapeDtypeStruct((NC, NP, D), jnp.float32),
        mesh=mesh,
        scratch_types=[
            pltpu.VMEM((CH,), jnp.int32),
            pltpu.VMEM((CH,), jnp.int32),
            pltpu.VMEM((CH, D), jnp.float32),
            pltpu.VMEM_SHARED((NP, D), jnp.float32),
            pltpu.SemaphoreType.DMA,
        ],
    )
    def _sc_agg(x_hbm, src_hbm, dst_hbm, zero_hbm, out_hbm,
                src_v, dst_v, rows_v, acc, sem):
        c = lax.axis_index("c")
        s = lax.axis_index("s")
        wid = s * NC + c
        pltpu.sync_copy(zero_hbm, acc.at[pl.ds(s * RPS, RPS)])
        plsc.subcore_barrier()

        def body(i, carry):
            base = wid * EPW + i * CH
            pltpu.sync_copy(src_hbm.at[pl.ds(base, CH)], src_v)
            pltpu.sync_copy(dst_hbm.at[pl.ds(base, CH)], dst_v)
            pltpu.async_copy(x_hbm.at[src_v], rows_v, sem).wait()
            pltpu.sync_copy(rows_v, acc.at[dst_v], add=True)
            return carry

        lax.fori_loop(0, NCHUNK, body, 0)
        plsc.subcore_barrier()
        pltpu.sync_copy(acc.at[pl.ds(s * RPS, RPS)],
                        out_hbm.at[c, pl.ds(s * RPS, RPS)])

    return _sc_agg


_PREC = jax.lax.Precision.HIGHEST


def _layer_math(x, agg, eps_s, W, b, gamma, beta):
    z = (1.0 + eps_s) * x + agg
    z = lax.dot(z, W, precision=_PREC, preferred_element_type=jnp.float32) + b
    mean = jnp.mean(z, axis=0, keepdims=True)
    var = jnp.mean((z - mean) ** 2, axis=0, keepdims=True)
    zn = (z - mean) * lax.rsqrt(var + 1e-5)
    return jnp.maximum(gamma * zn + beta, 0.0)


def _tc_layer0_body(x_ref, parts_ref, W_ref, b_ref, eps_ref, gamma_ref,
                    beta_ref, Wp0_ref, bp0_ref, Wp1_ref, bp1_ref,
                    x_out_ref, score_out_ref):
    x = x_ref[...]
    agg = parts_ref[0, :N, :] + parts_ref[1, :N, :]
    xn = _layer_math(x, agg, eps_ref[0, 0], W_ref[...], b_ref[...],
                     gamma_ref[...], beta_ref[...])
    x_out_ref[...] = xn
    score = lax.dot(x, Wp0_ref[...], precision=_PREC,
                    preferred_element_type=jnp.float32) + bp0_ref[...]
    score += lax.dot(xn, Wp1_ref[...], precision=_PREC,
                     preferred_element_type=jnp.float32) + bp1_ref[...]
    score_out_ref[...] = score


def _tc_layer_body(x_ref, parts_ref, W_ref, b_ref, eps_ref, gamma_ref,
                   beta_ref, Wp_ref, bp_ref, score_ref,
                   x_out_ref, score_out_ref):
    x = x_ref[...]
    agg = parts_ref[0, :N, :] + parts_ref[1, :N, :]
    xn = _layer_math(x, agg, eps_ref[0, 0], W_ref[...], b_ref[...],
                     gamma_ref[...], beta_ref[...])
    x_out_ref[...] = xn
    score_out_ref[...] = score_ref[...] + lax.dot(
        xn, Wp_ref[...], precision=_PREC,
        preferred_element_type=jnp.float32) + bp_ref[...]


_OUT_XS = [
    jax.ShapeDtypeStruct((N, D), jnp.float32),
    jax.ShapeDtypeStruct((N, C), jnp.float32),
]

_TC_PARAMS = pltpu.CompilerParams(vmem_limit_bytes=100 * 1024 * 1024)
_tc_layer0 = pl.pallas_call(_tc_layer0_body, out_shape=_OUT_XS,
                            compiler_params=_TC_PARAMS)
_tc_layer = pl.pallas_call(_tc_layer_body, out_shape=_OUT_XS,
                           compiler_params=_TC_PARAMS)


def kernel(h, edge_index, e, W, b, eps, gamma, beta, Wp, bp):
    del e
    src = edge_index[0]
    dst = edge_index[1]
    zblk = jnp.zeros((RPS, D), jnp.float32)
    eps2 = eps.reshape(L, 1, 1)
    b2 = b.reshape(L, 1, D)
    gamma2 = gamma.reshape(L, 1, D)
    beta2 = beta.reshape(L, 1, D)
    bp2 = bp.reshape(L + 1, 1, C)

    sc_agg = _make_sc_agg()
    x = h
    score = None
    for i in range(L):
        parts = sc_agg(x, src, dst, zblk)
        if i == 0:
            x, score = _tc_layer0(x, parts, W[0], b2[0], eps2[0], gamma2[0],
                                  beta2[0], Wp[0], bp2[0], Wp[1], bp2[1])
        else:
            x, score = _tc_layer(x, parts, W[i], b2[i], eps2[i], gamma2[i],
                                 beta2[i], Wp[i + 1], bp2[i + 1], score)
    return score


# speedup vs baseline: 9.3568x; 2.1967x over previous
import functools

import jax
import jax.numpy as jnp
from jax import lax
from jax.experimental import pallas as pl
from jax.experimental.pallas import tpu as pltpu
from jax.experimental.pallas import tpu_sc as plsc

N = 10000
E = 320000
D = 128
L = 4
C = 10

try:
    _info = plsc.get_sparse_core_info()
    NC = _info.num_cores
    NS = _info.num_subcores
except Exception:
    NC, NS = 2, 16
NW = NC * NS
EPW = E // NW
CH = 80
NCHUNK = EPW // CH
NPAIR = NCHUNK // 2
NP = 10240
RPS = NP // NS

@functools.cache
def _make_sc_agg():
    mesh = plsc.VectorSubcoreMesh(core_axis_name="c", subcore_axis_name="s")

    @functools.partial(
        pl.kernel,
        out_type=jax.ShapeDtypeStruct((NC, NP, D), jnp.float32),
        mesh=mesh,
        scratch_types=[
            pltpu.VMEM((EPW,), jnp.int32),
            pltpu.VMEM((CH,), jnp.int32),
            pltpu.VMEM((CH,), jnp.int32),
            pltpu.VMEM((CH, D), jnp.float32),
            pltpu.VMEM((CH, D), jnp.float32),
            pltpu.VMEM_SHARED((NP, D), jnp.float32),
            pltpu.SemaphoreType.DMA,
            pltpu.SemaphoreType.DMA,
            pltpu.SemaphoreType.DMA,
            pltpu.SemaphoreType.DMA,
            pltpu.SemaphoreType.DMA,
            pltpu.SemaphoreType.DMA,
        ],
    )
    def _sc_agg(x_hbm, src_hbm, dst_hbm, zero_hbm, out_hbm,
                src_sl, dstv0, dstv1, rows0, rows1, acc,
                sg0, sg1, ss0, ss1, sd0, sd1):
        c = lax.axis_index("c")
        s = lax.axis_index("s")
        wid = s * NC + c
        ebase = wid * EPW
        pltpu.sync_copy(src_hbm.at[pl.ds(ebase, EPW)], src_sl)
        pltpu.sync_copy(zero_hbm, acc.at[pl.ds(s * RPS, RPS)])
        plsc.subcore_barrier()

        def gather(i, buf, sem):
            pltpu.async_copy(x_hbm.at[src_sl.at[pl.ds(i * CH, CH)]], buf, sem)

        def wait_gather(buf, sem):
            pltpu.make_async_copy(x_hbm.at[src_sl.at[pl.ds(0, CH)]],
                                  buf, sem).wait()

        def scatter(buf, dstv, sem):
            pltpu.async_copy(buf, acc.at[dstv], sem, add=True)

        def wait_scatter(buf, dstv, sem):
            pltpu.make_async_copy(buf, acc.at[dstv], sem).wait()

        def load_dst(i, dstv, sem):
            pltpu.async_copy(dst_hbm.at[pl.ds(ebase + i * CH, CH)], dstv, sem)

        def wait_dst(dstv, sem):
            pltpu.make_async_copy(dst_hbm.at[pl.ds(0, CH)], dstv, sem).wait()

        load_dst(0, dstv0, sd0)
        load_dst(1, dstv1, sd1)
        gather(0, rows0, sg0)

        def body(k, carry):
            i0 = 2 * k
            i1 = i0 + 1

            @pl.when(k > 0)
            def _():
                wait_scatter(rows1, dstv1, ss1)
                load_dst(i1, dstv1, sd1)
            gather(i1, rows1, sg1)
            wait_gather(rows0, sg0)
            wait_dst(dstv0, sd0)
            scatter(rows0, dstv0, ss0)

            @pl.when(k < NPAIR - 1)
            def _():
                wait_scatter(rows0, dstv0, ss0)
                load_dst(i0 + 2, dstv0, sd0)
                gather(i0 + 2, rows0, sg0)
            wait_gather(rows1, sg1)
            wait_dst(dstv1, sd1)
            scatter(rows1, dstv1, ss1)
            return carry

        lax.fori_loop(0, NPAIR, body, 0)
        wait_scatter(rows0, dstv0, ss0)
        wait_scatter(rows1, dstv1, ss1)
        load_dst(NCHUNK - 1, dstv0, sd0)
        gather(NCHUNK - 1, rows0, sg0)
        wait_dst(dstv0, sd0)
        wait_gather(rows0, sg0)
        scatter(rows0, dstv0, ss0)
        wait_scatter(rows0, dstv0, ss0)
        plsc.subcore_barrier()
        pltpu.sync_copy(acc.at[pl.ds(s * RPS, RPS)],
                        out_hbm.at[c, pl.ds(s * RPS, RPS)])

    return _sc_agg


_PREC = jax.lax.Precision.HIGHEST


def _layer_math(x, agg, eps_s, W, b, gamma, beta):
    z = (1.0 + eps_s) * x + agg
    z = lax.dot(z, W, precision=_PREC, preferred_element_type=jnp.float32) + b
    mean = jnp.mean(z, axis=0, keepdims=True)
    var = jnp.mean((z - mean) ** 2, axis=0, keepdims=True)
    zn = (z - mean) * lax.rsqrt(var + 1e-5)
    return jnp.maximum(gamma * zn + beta, 0.0)


def _tc_layer0_body(x_ref, parts_ref, W_ref, b_ref, eps_ref, gamma_ref,
                    beta_ref, Wp0_ref, bp0_ref, Wp1_ref, bp1_ref,
                    x_out_ref, score_out_ref):
    x = x_ref[...]
    agg = parts_ref[0, :N, :] + parts_ref[1, :N, :]
    xn = _layer_math(x, agg, eps_ref[0, 0], W_ref[...], b_ref[...],
                     gamma_ref[...], beta_ref[...])
    x_out_ref[...] = xn
    score = lax.dot(x, Wp0_ref[...], precision=_PREC,
                    preferred_element_type=jnp.float32) + bp0_ref[...]
    score += lax.dot(xn, Wp1_ref[...], precision=_PREC,
                     preferred_element_type=jnp.float32) + bp1_ref[...]
    score_out_ref[...] = score


def _tc_layer_body(x_ref, parts_ref, W_ref, b_ref, eps_ref, gamma_ref,
                   beta_ref, Wp_ref, bp_ref, score_ref,
                   x_out_ref, score_out_ref):
    x = x_ref[...]
    agg = parts_ref[0, :N, :] + parts_ref[1, :N, :]
    xn = _layer_math(x, agg, eps_ref[0, 0], W_ref[...], b_ref[...],
                     gamma_ref[...], beta_ref[...])
    x_out_ref[...] = xn
    score_out_ref[...] = score_ref[...] + lax.dot(
        xn, Wp_ref[...], precision=_PREC,
        preferred_element_type=jnp.float32) + bp_ref[...]


_OUT_XS = [
    jax.ShapeDtypeStruct((N, D), jnp.float32),
    jax.ShapeDtypeStruct((N, C), jnp.float32),
]

_TC_PARAMS = pltpu.CompilerParams(vmem_limit_bytes=100 * 1024 * 1024)
_tc_layer0 = pl.pallas_call(_tc_layer0_body, out_shape=_OUT_XS,
                            compiler_params=_TC_PARAMS)
_tc_layer = pl.pallas_call(_tc_layer_body, out_shape=_OUT_XS,
                           compiler_params=_TC_PARAMS)


def kernel(h, edge_index, e, W, b, eps, gamma, beta, Wp, bp):
    del e
    src = edge_index[0]
    dst = edge_index[1]
    zblk = jnp.zeros((RPS, D), jnp.float32)
    eps2 = eps.reshape(L, 1, 1)
    b2 = b.reshape(L, 1, D)
    gamma2 = gamma.reshape(L, 1, D)
    beta2 = beta.reshape(L, 1, D)
    bp2 = bp.reshape(L + 1, 1, C)

    sc_agg = _make_sc_agg()
    x = h
    score = None
    for i in range(L):
        parts = sc_agg(x, src, dst, zblk)
        if i == 0:
            x, score = _tc_layer0(x, parts, W[0], b2[0], eps2[0], gamma2[0],
                                  beta2[0], Wp[0], bp2[0], Wp[1], bp2[1])
        else:
            x, score = _tc_layer(x, parts, W[i], b2[i], eps2[i], gamma2[i],
                                 beta2[i], Wp[i + 1], bp2[i + 1], score)
    return score
